# baseline (speedup 1.0000x reference)
import functools

import jax
import jax.numpy as jnp
from jax import lax
from jax.experimental import pallas as pl
from jax.experimental.pallas import tpu as pltpu
from jax.experimental.pallas import tpu_sc as plsc

N = 10000
E = 320000
D = 128
E_CHARGE = 1.602176634e-19
EPS0 = 8.8541878128e-12

NC = 2
NS = 16
NW = NC * NS
EPW = E // NW
CH = 80
NCHUNK = EPW // CH
NPAD = 10240
RPT = NPAD // NS
ZR = 64

_C_E2 = E_CHARGE ** 2
_C_4PIEPS = 4.0 * 3.141592653589793 * EPS0
_C_1EM10 = 1e-10


def _mish(v):
    return v * jnp.tanh(jax.nn.softplus(v))


def _mlp(v, w1, b1, w2, b2):
    h = jnp.dot(v, w1, preferred_element_type=jnp.float32) + b1
    return jnp.dot(_mish(h), w2, preferred_element_type=jnp.float32) + b2


BN = 2000


def _node_body(x_ref, ws1, bs1, ws2, bs2, wd1, bd1, wd2, bd2, hs_ref, hd_ref):
    xb = x_ref[...]
    hs_ref[...] = _mlp(xb, ws1[...], bs1[...], ws2[...], bs2[...])
    hd_ref[...] = _mlp(xb, wd1[...], bd1[...], wd2[...], bd2[...])


def _node_mlps(x, ws1, bs1, ws2, bs2, wd1, bd1, wd2, bd2):
    wspec = pl.BlockSpec((D, D), lambda i: (0, 0))
    bspec = pl.BlockSpec((1, D), lambda i: (0, 0))
    nspec = pl.BlockSpec((BN, D), lambda i: (i, 0))
    return pl.pallas_call(
        _node_body,
        grid=(N // BN,),
        in_specs=[nspec, wspec, bspec, wspec, bspec, wspec, bspec, wspec, bspec],
        out_specs=[nspec, nspec],
        out_shape=[jax.ShapeDtypeStruct((N, D), jnp.float32)] * 2,
    )(x, ws1, bs1, ws2, bs2, wd1, bd1, wd2, bd2)


@functools.cache
def _sc_mesh():
    return plsc.VectorSubcoreMesh(
        core_axis_name="c", subcore_axis_name="s",
        num_cores=NC, num_subcores=NS)


@functools.cache
def _gather_mul_kernel():
    return functools.partial(
        pl.kernel,
        out_type=jax.ShapeDtypeStruct((E, D), jnp.float32),
        mesh=_sc_mesh(),
        scratch_types=[
            pltpu.VMEM((NCHUNK, CH), jnp.int32),
            pltpu.VMEM((NCHUNK, CH), jnp.int32),
            pltpu.VMEM((CH, D), jnp.float32),
            pltpu.VMEM((CH, D), jnp.float32),
            pltpu.VMEM((CH, D), jnp.float32),
            pltpu.VMEM((CH, D), jnp.float32),
            pltpu.VMEM((CH, D), jnp.float32),
            pltpu.VMEM((CH, D), jnp.float32),
            pltpu.SemaphoreType.DMA,
            pltpu.SemaphoreType.DMA,
            pltpu.SemaphoreType.DMA,
            pltpu.SemaphoreType.DMA,
        ],
    )(_gather_mul_body)


def _gather_mul_body(hsrc_hbm, hdst_hbm, src_hbm, dst_hbm, g_hbm,
                     ia_v, ib_v, ra0, rb0, ra1, rb1, wa0, wa1,
                     sem0, sem1, wsem0, wsem1):
    wid = lax.axis_index("s") * NC + lax.axis_index("c")
    base = wid * EPW
    pltpu.sync_copy(src_hbm.at[wid], ia_v)
    pltpu.sync_copy(dst_hbm.at[wid], ib_v)
    gbufs = ((ra0, rb0, sem0), (ra1, rb1, sem1))
    wbufs = ((wa0, wsem0), (wa1, wsem1))

    def fire(c, ra, rb, sem):
        pltpu.async_copy(hsrc_hbm.at[ia_v.at[c]], ra, sem)
        pltpu.async_copy(hdst_hbm.at[ib_v.at[c]], rb, sem)

    def process(c, p):
        ra, rb, sem = gbufs[p]
        wa, wsem = wbufs[p]
        pltpu.make_async_copy(hsrc_hbm.at[ia_v.at[c]], ra, sem).wait()
        pltpu.make_async_copy(hdst_hbm.at[ib_v.at[c]], rb, sem).wait()

        @pl.when(c >= 2)
        def _():
            pltpu.make_async_copy(
                wa, g_hbm.at[pl.ds(base + (c - 2) * CH, CH)], wsem).wait()

        @plsc.parallel_loop(0, CH, unroll=4)
        def _(i):
            for r in range(D // 16):
                sl = pl.ds(r * 16, 16)
                wa[i, sl] = ra[i, sl] * rb[i, sl]
        nc = c + 2

        @pl.when(nc < NCHUNK)
        def _():
            fire(nc, ra, rb, sem)

        pltpu.async_copy(wa, g_hbm.at[pl.ds(base + c * CH, CH)], wsem)

    fire(0, *gbufs[0])
    fire(1, *gbufs[1])

    def body(k, carry):
        for p in range(2):
            process(2 * k + p, p)
        return carry

    lax.fori_loop(0, NCHUNK // 2, body, 0)
    lastp = (NCHUNK - 1) % 2
    process(jnp.int32(NCHUNK - 1), lastp)
    for p in range(2):
        wa, wsem = wbufs[p]
        c = NCHUNK - 1 if p == lastp else NCHUNK - 2
        pltpu.make_async_copy(
            wa, g_hbm.at[pl.ds(base + c * CH, CH)], wsem).wait()


BE = 2000


def _ef_body(ef_ref, we1, be1, we2, be2, out_ref):
    out_ref[...] = _mlp(ef_ref[...], we1[...], be1[...], we2[...], be2[...])


def _ef_mlp(edge_feats, we1, be1, we2, be2):
    wspec = pl.BlockSpec((D, D), lambda i: (0, 0))
    bspec = pl.BlockSpec((1, D), lambda i: (0, 0))
    espec = pl.BlockSpec((BE, D), lambda i: (i, 0))
    return pl.pallas_call(
        _ef_body,
        grid=(E // BE,),
        in_specs=[espec, wspec, bspec, wspec, bspec],
        out_specs=espec,
        out_shape=jax.ShapeDtypeStruct((E, D), jnp.float32),
    )(edge_feats, we1, be1, we2, be2)


def _msg_body(ef_ref, g_ref, wm1, bm1, wm2, bm2, m_ref):
    num = g_ref[...] * _C_E2
    den = (_C_4PIEPS * ef_ref[...]) * _C_1EM10
    hn = num / den
    m_ref[...] = _mlp(hn, wm1[...], bm1[...], wm2[...], bm2[...])


def _msg_mlp(ef, g, wm1, bm1, wm2, bm2):
    wspec = pl.BlockSpec((D, D), lambda i: (0, 0))
    bspec = pl.BlockSpec((1, D), lambda i: (0, 0))
    espec = pl.BlockSpec((BE, D), lambda i: (i, 0))
    return pl.pallas_call(
        _msg_body,
        grid=(E // BE,),
        in_specs=[espec, espec, wspec, bspec, wspec, bspec],
        out_specs=espec,
        out_shape=jax.ShapeDtypeStruct((E, D), jnp.float32),
    )(ef, g, wm1, bm1, wm2, bm2)


@functools.cache
def _scatter_add_kernel():
    return functools.partial(
        pl.kernel,
        out_type=jax.ShapeDtypeStruct((NC * NPAD, D), jnp.float32),
        mesh=_sc_mesh(),
        scratch_types=[
            pltpu.VMEM((NCHUNK, CH), jnp.int32),
            pltpu.VMEM((CH, D), jnp.float32),
            pltpu.VMEM((CH, D), jnp.float32),
            pltpu.VMEM((ZR, D), jnp.float32),
            pltpu.VMEM_SHARED((NPAD, D), jnp.float32),
            pltpu.SemaphoreType.DMA,
            pltpu.SemaphoreType.DMA,
        ],
    )(_scatter_add_body)


def _scatter_add_body(m_hbm, dst_hbm, h_hbm, idx_v, rows0, rows1, zb_v,
                      shared, sem0, sem1):
    cid = lax.axis_index("c")
    sid = lax.axis_index("s")
    wid = sid * NC + cid
    base = wid * EPW
    pltpu.sync_copy(dst_hbm.at[wid], idx_v)

    def zrow(i, carry):
        for r in range(D // 16):
            zb_v[i, pl.ds(r * 16, 16)] = jnp.zeros((16,), jnp.float32)
        return carry

    lax.fori_loop(0, ZR, zrow, 0)
    for k in range(RPT // ZR):
        pltpu.sync_copy(zb_v, shared.at[pl.ds(sid * RPT + k * ZR, ZR)])
    plsc.subcore_barrier()

    bufs = ((rows0, sem0), (rows1, sem1))

    def fire(c, rows, sem):
        pltpu.async_copy(m_hbm.at[pl.ds(base + c * CH, CH)], rows, sem)

    def process(c, rows, sem):
        pltpu.make_async_copy(m_hbm.at[pl.ds(base + c * CH, CH)], rows,
                              sem).wait()
        pltpu.sync_copy(rows, shared.at[idx_v.at[c]], add=True)

    fire(0, *bufs[0])
    fire(1, *bufs[1])

    def chunk(k, carry):
        for p in range(2):
            c = 2 * k + p
            rows, sem = bufs[p]
            process(c, rows, sem)
            nc = c + 2

            @pl.when(nc < NCHUNK)
            def _():
                fire(nc, rows, sem)
        return carry

    lax.fori_loop(0, NCHUNK // 2, chunk, 0)
    process(NCHUNK - 1, *bufs[(NCHUNK - 1) % 2])
    plsc.subcore_barrier()
    pltpu.sync_copy(shared.at[pl.ds(sid * RPT, RPT)],
                    h_hbm.at[pl.ds(cid * NPAD + sid * RPT, RPT)])


def _final_body(x_ref, h0_ref, h1_ref, out_ref):
    out_ref[...] = jax.nn.softplus(x_ref[...] + (h0_ref[...] + h1_ref[...]))


def _finalize(x, h0, h1):
    nspec = pl.BlockSpec((BN, D), lambda i: (i, 0))
    return pl.pallas_call(
        _final_body,
        grid=(N // BN,),
        in_specs=[nspec, nspec, nspec],
        out_specs=nspec,
        out_shape=jax.ShapeDtypeStruct((N, D), jnp.float32),
    )(x, h0, h1)


def kernel(x, edge_index, edge_feats,
           W_src_1, b_src_1, W_src_2, b_src_2,
           W_dst_1, b_dst_1, W_dst_2, b_dst_2,
           W_edge_1, b_edge_1, W_edge_2, b_edge_2,
           W_m_1, b_m_1, W_m_2, b_m_2):
    src = edge_index[0].reshape(NW, NCHUNK, CH)
    dst = edge_index[1].reshape(NW, NCHUNK, CH)
    b2 = lambda b: b.reshape(1, D)
    h_src, h_dst = _node_mlps(x, W_src_1, b2(b_src_1), W_src_2, b2(b_src_2),
                              W_dst_1, b2(b_dst_1), W_dst_2, b2(b_dst_2))
    ef = _ef_mlp(edge_feats, W_edge_1, b2(b_edge_1), W_edge_2, b2(b_edge_2))
    g = _gather_mul_kernel()(h_src, h_dst, src, dst)
    m = _msg_mlp(ef, g, W_m_1, b2(b_m_1), W_m_2, b2(b_m_2))
    h_parts = _scatter_add_kernel()(m, dst)
    return _finalize(x, h_parts[:N], h_parts[NPAD:NPAD + N])


# speedup vs baseline: 1.0606x; 1.0606x over previous
import functools

import jax
import jax.numpy as jnp
from jax import lax
from jax.experimental import pallas as pl
from jax.experimental.pallas import tpu as pltpu
from jax.experimental.pallas import tpu_sc as plsc

N = 10000
E = 320000
D = 128
E_CHARGE = 1.602176634e-19
EPS0 = 8.8541878128e-12

NC = 2
NS = 16
NW = NC * NS
EPW = E // NW
CH = 80
NCHUNK = EPW // CH
NPAD = 10240
RPT = NPAD // NS
ZR = 64

_C_E2 = E_CHARGE ** 2
_C_4PIEPS = 4.0 * 3.141592653589793 * EPS0
_C_1EM10 = 1e-10


def _mish(v):
    return v * jnp.tanh(jax.nn.softplus(v))


def _mlp(v, w1, b1, w2, b2):
    h = jnp.dot(v, w1, preferred_element_type=jnp.float32) + b1
    return jnp.dot(_mish(h), w2, preferred_element_type=jnp.float32) + b2


BN = 1000


def _node_body(x_ref, ws1, bs1, ws2, bs2, wd1, bd1, wd2, bd2, hs_ref, hd_ref):
    xb = x_ref[...]
    hs_ref[...] = _mlp(xb, ws1[...], bs1[...], ws2[...], bs2[...])
    hd_ref[...] = _mlp(xb, wd1[...], bd1[...], wd2[...], bd2[...])


def _node_mlps(x, ws1, bs1, ws2, bs2, wd1, bd1, wd2, bd2):
    wspec = pl.BlockSpec((D, D), lambda i: (0, 0))
    bspec = pl.BlockSpec((1, D), lambda i: (0, 0))
    nspec = pl.BlockSpec((BN, D), lambda i: (i, 0))
    return pl.pallas_call(
        _node_body,
        grid=(N // BN,),
        in_specs=[nspec, wspec, bspec, wspec, bspec, wspec, bspec, wspec, bspec],
        out_specs=[nspec, nspec],
        out_shape=[jax.ShapeDtypeStruct((N, D), jnp.float32)] * 2,
    )(x, ws1, bs1, ws2, bs2, wd1, bd1, wd2, bd2)


@functools.cache
def _sc_mesh():
    return plsc.VectorSubcoreMesh(
        core_axis_name="c", subcore_axis_name="s",
        num_cores=NC, num_subcores=NS)


@functools.cache
def _gather_mul_kernel():
    return functools.partial(
        pl.kernel,
        out_type=jax.ShapeDtypeStruct((E, D), jnp.float32),
        mesh=_sc_mesh(),
        scratch_types=[
            pltpu.VMEM((NCHUNK, CH), jnp.int32),
            pltpu.VMEM((NCHUNK, CH), jnp.int32),
            pltpu.VMEM((CH, D), jnp.float32),
            pltpu.VMEM((CH, D), jnp.float32),
            pltpu.VMEM((CH, D), jnp.float32),
            pltpu.VMEM((CH, D), jnp.float32),
            pltpu.VMEM((CH, D), jnp.float32),
            pltpu.VMEM((CH, D), jnp.float32),
            pltpu.SemaphoreType.DMA,
            pltpu.SemaphoreType.DMA,
            pltpu.SemaphoreType.DMA,
            pltpu.SemaphoreType.DMA,
        ],
    )(_gather_mul_body)


def _gather_mul_body(hsrc_hbm, hdst_hbm, src_hbm, dst_hbm, g_hbm,
                     ia_v, ib_v, ra0, rb0, ra1, rb1, wa0, wa1,
                     sem0, sem1, wsem0, wsem1):
    wid = lax.axis_index("s") * NC + lax.axis_index("c")
    base = wid * EPW
    pltpu.sync_copy(src_hbm.at[wid], ia_v)
    pltpu.sync_copy(dst_hbm.at[wid], ib_v)
    gbufs = ((ra0, rb0, sem0), (ra1, rb1, sem1))
    wbufs = ((wa0, wsem0), (wa1, wsem1))

    def fire(c, ra, rb, sem):
        pltpu.async_copy(hsrc_hbm.at[ia_v.at[c]], ra, sem)
        pltpu.async_copy(hdst_hbm.at[ib_v.at[c]], rb, sem)

    def process(c, p):
        ra, rb, sem = gbufs[p]
        wa, wsem = wbufs[p]
        pltpu.make_async_copy(hsrc_hbm.at[ia_v.at[c]], ra, sem).wait()
        pltpu.make_async_copy(hdst_hbm.at[ib_v.at[c]], rb, sem).wait()

        @pl.when(c >= 2)
        def _():
            pltpu.make_async_copy(
                wa, g_hbm.at[pl.ds(base + (c - 2) * CH, CH)], wsem).wait()

        @plsc.parallel_loop(0, CH, unroll=4)
        def _(i):
            for r in range(D // 16):
                sl = pl.ds(r * 16, 16)
                wa[i, sl] = ra[i, sl] * rb[i, sl]
        nc = c + 2

        @pl.when(nc < NCHUNK)
        def _():
            fire(nc, ra, rb, sem)

        pltpu.async_copy(wa, g_hbm.at[pl.ds(base + c * CH, CH)], wsem)

    fire(0, *gbufs[0])
    fire(1, *gbufs[1])

    def body(k, carry):
        for p in range(2):
            process(2 * k + p, p)
        return carry

    lax.fori_loop(0, NCHUNK // 2, body, 0)
    lastp = (NCHUNK - 1) % 2
    process(jnp.int32(NCHUNK - 1), lastp)
    for p in range(2):
        wa, wsem = wbufs[p]
        c = NCHUNK - 1 if p == lastp else NCHUNK - 2
        pltpu.make_async_copy(
            wa, g_hbm.at[pl.ds(base + c * CH, CH)], wsem).wait()


BE = 2000


def _edge_body(ef_ref, g_ref, we1, be1, we2, be2, wm1, bm1, wm2, bm2, m_ref):
    ef = _mlp(ef_ref[...], we1[...], be1[...], we2[...], be2[...])
    num = g_ref[...] * _C_E2
    den = (_C_4PIEPS * ef) * _C_1EM10
    hn = num / den
    m_ref[...] = _mlp(hn, wm1[...], bm1[...], wm2[...], bm2[...])


def _edge_mlps(edge_feats, g, we1, be1, we2, be2, wm1, bm1, wm2, bm2):
    wspec = pl.BlockSpec((D, D), lambda i: (0, 0))
    bspec = pl.BlockSpec((1, D), lambda i: (0, 0))
    espec = pl.BlockSpec((BE, D), lambda i: (i, 0))
    return pl.pallas_call(
        _edge_body,
        grid=(E // BE,),
        in_specs=[espec, espec, wspec, bspec, wspec, bspec,
                  wspec, bspec, wspec, bspec],
        out_specs=espec,
        out_shape=jax.ShapeDtypeStruct((E, D), jnp.float32),
    )(edge_feats, g, we1, be1, we2, be2, wm1, bm1, wm2, bm2)


@functools.cache
def _scatter_add_kernel():
    return functools.partial(
        pl.kernel,
        out_type=jax.ShapeDtypeStruct((NC * NPAD, D), jnp.float32),
        mesh=_sc_mesh(),
        scratch_types=[
            pltpu.VMEM((NCHUNK, CH), jnp.int32),
            pltpu.VMEM((CH, D), jnp.float32),
            pltpu.VMEM((CH, D), jnp.float32),
            pltpu.VMEM((ZR, D), jnp.float32),
            pltpu.VMEM_SHARED((NPAD, D), jnp.float32),
            pltpu.SemaphoreType.DMA,
            pltpu.SemaphoreType.DMA,
        ],
    )(_scatter_add_body)


def _scatter_add_body(m_hbm, dst_hbm, h_hbm, idx_v, rows0, rows1, zb_v,
                      shared, sem0, sem1):
    cid = lax.axis_index("c")
    sid = lax.axis_index("s")
    wid = sid * NC + cid
    base = wid * EPW
    pltpu.sync_copy(dst_hbm.at[wid], idx_v)

    def zrow(i, carry):
        for r in range(D // 16):
            zb_v[i, pl.ds(r * 16, 16)] = jnp.zeros((16,), jnp.float32)
        return carry

    lax.fori_loop(0, ZR, zrow, 0)
    for k in range(RPT // ZR):
        pltpu.sync_copy(zb_v, shared.at[pl.ds(sid * RPT + k * ZR, ZR)])
    plsc.subcore_barrier()

    bufs = ((rows0, sem0), (rows1, sem1))

    def fire(c, rows, sem):
        pltpu.async_copy(m_hbm.at[pl.ds(base + c * CH, CH)], rows, sem)

    def process(c, rows, sem):
        pltpu.make_async_copy(m_hbm.at[pl.ds(base + c * CH, CH)], rows,
                              sem).wait()
        pltpu.sync_copy(rows, shared.at[idx_v.at[c]], add=True)

    fire(0, *bufs[0])
    fire(1, *bufs[1])

    def chunk(k, carry):
        for p in range(2):
            c = 2 * k + p
            rows, sem = bufs[p]
            process(c, rows, sem)
            nc = c + 2

            @pl.when(nc < NCHUNK)
            def _():
                fire(nc, rows, sem)
        return carry

    lax.fori_loop(0, NCHUNK // 2, chunk, 0)
    process(NCHUNK - 1, *bufs[(NCHUNK - 1) % 2])
    plsc.subcore_barrier()
    pltpu.sync_copy(shared.at[pl.ds(sid * RPT, RPT)],
                    h_hbm.at[pl.ds(cid * NPAD + sid * RPT, RPT)])


def _final_body(x_ref, h0_ref, h1_ref, out_ref):
    out_ref[...] = jax.nn.softplus(x_ref[...] + (h0_ref[...] + h1_ref[...]))


def _finalize(x, h0, h1):
    nspec = pl.BlockSpec((BN, D), lambda i: (i, 0))
    return pl.pallas_call(
        _final_body,
        grid=(N // BN,),
        in_specs=[nspec, nspec, nspec],
        out_specs=nspec,
        out_shape=jax.ShapeDtypeStruct((N, D), jnp.float32),
    )(x, h0, h1)


def kernel(x, edge_index, edge_feats,
           W_src_1, b_src_1, W_src_2, b_src_2,
           W_dst_1, b_dst_1, W_dst_2, b_dst_2,
           W_edge_1, b_edge_1, W_edge_2, b_edge_2,
           W_m_1, b_m_1, W_m_2, b_m_2):
    src = edge_index[0].reshape(NW, NCHUNK, CH)
    dst = edge_index[1].reshape(NW, NCHUNK, CH)
    b2 = lambda b: b.reshape(1, D)
    h_src, h_dst = _node_mlps(x, W_src_1, b2(b_src_1), W_src_2, b2(b_src_2),
                              W_dst_1, b2(b_dst_1), W_dst_2, b2(b_dst_2))
    g = _gather_mul_kernel()(h_src, h_dst, src, dst)
    m = _edge_mlps(edge_feats, g, W_edge_1, b2(b_edge_1), W_edge_2, b2(b_edge_2),
                   W_m_1, b2(b_m_1), W_m_2, b2(b_m_2))
    h_parts = _scatter_add_kernel()(m, dst)
    return _finalize(x, h_parts[:N], h_parts[NPAD:NPAD + N])


# speedup vs baseline: 1.1475x; 1.0819x over previous
import functools

import jax
import jax.numpy as jnp
from jax import lax
from jax.experimental import pallas as pl
from jax.experimental.pallas import tpu as pltpu
from jax.experimental.pallas import tpu_sc as plsc

N = 10000
E = 320000
D = 128
E_CHARGE = 1.602176634e-19
EPS0 = 8.8541878128e-12

NC = 2
NS = 16
NW = NC * NS
EPW = E // NW
CH = 80
NCHUNK = EPW // CH
NPAD = 10240
RPT = NPAD // NS
ZR = 64

_C_E2 = E_CHARGE ** 2
_C_4PIEPS = 4.0 * 3.141592653589793 * EPS0
_C_1EM10 = 1e-10


def _mish(v):
    u = jnp.exp(jnp.minimum(v, 30.0))
    t = u * (u + 2.0)
    return v * (t / (t + 2.0))


def _mlp(v, w1, b1, w2, b2):
    h = jnp.dot(v, w1, preferred_element_type=jnp.float32) + b1
    return jnp.dot(_mish(h), w2, preferred_element_type=jnp.float32) + b2


BN = 1000


def _node_body(x_ref, ws1, bs1, ws2, bs2, wd1, bd1, wd2, bd2, hs_ref, hd_ref):
    xb = x_ref[...]
    hs_ref[...] = _mlp(xb, ws1[...], bs1[...], ws2[...], bs2[...])
    hd_ref[...] = _mlp(xb, wd1[...], bd1[...], wd2[...], bd2[...])


def _node_mlps(x, ws1, bs1, ws2, bs2, wd1, bd1, wd2, bd2):
    wspec = pl.BlockSpec((D, D), lambda i: (0, 0))
    bspec = pl.BlockSpec((1, D), lambda i: (0, 0))
    nspec = pl.BlockSpec((BN, D), lambda i: (i, 0))
    return pl.pallas_call(
        _node_body,
        grid=(N // BN,),
        in_specs=[nspec, wspec, bspec, wspec, bspec, wspec, bspec, wspec, bspec],
        out_specs=[nspec, nspec],
        out_shape=[jax.ShapeDtypeStruct((N, D), jnp.float32)] * 2,
    )(x, ws1, bs1, ws2, bs2, wd1, bd1, wd2, bd2)


@functools.cache
def _sc_mesh():
    return plsc.VectorSubcoreMesh(
        core_axis_name="c", subcore_axis_name="s",
        num_cores=NC, num_subcores=NS)


@functools.cache
def _gather_mul_kernel():
    return functools.partial(
        pl.kernel,
        out_type=jax.ShapeDtypeStruct((E, D), jnp.float32),
        mesh=_sc_mesh(),
        scratch_types=[
            pltpu.VMEM((NCHUNK, CH), jnp.int32),
            pltpu.VMEM((NCHUNK, CH), jnp.int32),
            pltpu.VMEM((CH, D), jnp.float32),
            pltpu.VMEM((CH, D), jnp.float32),
            pltpu.VMEM((CH, D), jnp.float32),
            pltpu.VMEM((CH, D), jnp.float32),
            pltpu.VMEM((CH, D), jnp.float32),
            pltpu.VMEM((CH, D), jnp.float32),
            pltpu.SemaphoreType.DMA,
            pltpu.SemaphoreType.DMA,
            pltpu.SemaphoreType.DMA,
            pltpu.SemaphoreType.DMA,
        ],
    )(_gather_mul_body)


def _gather_mul_body(hsrc_hbm, hdst_hbm, src_hbm, dst_hbm, g_hbm,
                     ia_v, ib_v, ra0, rb0, ra1, rb1, wa0, wa1,
                     sem0, sem1, wsem0, wsem1):
    wid = lax.axis_index("s") * NC + lax.axis_index("c")
    base = wid * EPW
    pltpu.sync_copy(src_hbm.at[wid], ia_v)
    pltpu.sync_copy(dst_hbm.at[wid], ib_v)
    gbufs = ((ra0, rb0, sem0), (ra1, rb1, sem1))
    wbufs = ((wa0, wsem0), (wa1, wsem1))

    def fire(c, ra, rb, sem):
        pltpu.async_copy(hsrc_hbm.at[ia_v.at[c]], ra, sem)
        pltpu.async_copy(hdst_hbm.at[ib_v.at[c]], rb, sem)

    def process(c, p):
        ra, rb, sem = gbufs[p]
        wa, wsem = wbufs[p]
        pltpu.make_async_copy(hsrc_hbm.at[ia_v.at[c]], ra, sem).wait()
        pltpu.make_async_copy(hdst_hbm.at[ib_v.at[c]], rb, sem).wait()

        @pl.when(c >= 2)
        def _():
            pltpu.make_async_copy(
                wa, g_hbm.at[pl.ds(base + (c - 2) * CH, CH)], wsem).wait()

        @plsc.parallel_loop(0, CH, unroll=4)
        def _(i):
            for r in range(D // 16):
                sl = pl.ds(r * 16, 16)
                wa[i, sl] = ra[i, sl] * rb[i, sl]
        nc = c + 2

        @pl.when(nc < NCHUNK)
        def _():
            fire(nc, ra, rb, sem)

        pltpu.async_copy(wa, g_hbm.at[pl.ds(base + c * CH, CH)], wsem)

    fire(0, *gbufs[0])
    fire(1, *gbufs[1])

    def body(k, carry):
        for p in range(2):
            process(2 * k + p, p)
        return carry

    lax.fori_loop(0, NCHUNK // 2, body, 0)
    lastp = (NCHUNK - 1) % 2
    process(jnp.int32(NCHUNK - 1), lastp)
    for p in range(2):
        wa, wsem = wbufs[p]
        c = NCHUNK - 1 if p == lastp else NCHUNK - 2
        pltpu.make_async_copy(
            wa, g_hbm.at[pl.ds(base + c * CH, CH)], wsem).wait()


BE = 2000


def _edge_body(ef_ref, g_ref, we1, be1, we2, be2, wm1, bm1, wm2, bm2, m_ref):
    ef = _mlp(ef_ref[...], we1[...], be1[...], we2[...], be2[...])
    num = g_ref[...] * _C_E2
    den = (_C_4PIEPS * ef) * _C_1EM10
    hn = num / den
    m_ref[...] = _mlp(hn, wm1[...], bm1[...], wm2[...], bm2[...])


def _edge_mlps(edge_feats, g, we1, be1, we2, be2, wm1, bm1, wm2, bm2):
    wspec = pl.BlockSpec((D, D), lambda i: (0, 0))
    bspec = pl.BlockSpec((1, D), lambda i: (0, 0))
    espec = pl.BlockSpec((BE, D), lambda i: (i, 0))
    return pl.pallas_call(
        _edge_body,
        grid=(E // BE,),
        in_specs=[espec, espec, wspec, bspec, wspec, bspec,
                  wspec, bspec, wspec, bspec],
        out_specs=espec,
        out_shape=jax.ShapeDtypeStruct((E, D), jnp.float32),
    )(edge_feats, g, we1, be1, we2, be2, wm1, bm1, wm2, bm2)


@functools.cache
def _scatter_add_kernel():
    return functools.partial(
        pl.kernel,
        out_type=jax.ShapeDtypeStruct((NC * NPAD, D), jnp.float32),
        mesh=_sc_mesh(),
        scratch_types=[
            pltpu.VMEM((NCHUNK, CH), jnp.int32),
            pltpu.VMEM((CH, D), jnp.float32),
            pltpu.VMEM((CH, D), jnp.float32),
            pltpu.VMEM((ZR, D), jnp.float32),
            pltpu.VMEM_SHARED((NPAD, D), jnp.float32),
            pltpu.SemaphoreType.DMA,
            pltpu.SemaphoreType.DMA,
        ],
    )(_scatter_add_body)


def _scatter_add_body(m_hbm, dst_hbm, h_hbm, idx_v, rows0, rows1, zb_v,
                      shared, sem0, sem1):
    cid = lax.axis_index("c")
    sid = lax.axis_index("s")
    wid = sid * NC + cid
    base = wid * EPW
    pltpu.sync_copy(dst_hbm.at[wid], idx_v)

    def zrow(i, carry):
        for r in range(D // 16):
            zb_v[i, pl.ds(r * 16, 16)] = jnp.zeros((16,), jnp.float32)
        return carry

    lax.fori_loop(0, ZR, zrow, 0)
    for k in range(RPT // ZR):
        pltpu.sync_copy(zb_v, shared.at[pl.ds(sid * RPT + k * ZR, ZR)])
    plsc.subcore_barrier()

    bufs = ((rows0, sem0), (rows1, sem1))

    def fire(c, rows, sem):
        pltpu.async_copy(m_hbm.at[pl.ds(base + c * CH, CH)], rows, sem)

    def process(c, rows, sem):
        pltpu.make_async_copy(m_hbm.at[pl.ds(base + c * CH, CH)], rows,
                              sem).wait()
        pltpu.sync_copy(rows, shared.at[idx_v.at[c]], add=True)

    fire(0, *bufs[0])
    fire(1, *bufs[1])

    def chunk(k, carry):
        for p in range(2):
            c = 2 * k + p
            rows, sem = bufs[p]
            process(c, rows, sem)
            nc = c + 2

            @pl.when(nc < NCHUNK)
            def _():
                fire(nc, rows, sem)
        return carry

    lax.fori_loop(0, NCHUNK // 2, chunk, 0)
    process(NCHUNK - 1, *bufs[(NCHUNK - 1) % 2])
    plsc.subcore_barrier()
    pltpu.sync_copy(shared.at[pl.ds(sid * RPT, RPT)],
                    h_hbm.at[pl.ds(cid * NPAD + sid * RPT, RPT)])


def _final_body(x_ref, h0_ref, h1_ref, out_ref):
    out_ref[...] = jax.nn.softplus(x_ref[...] + (h0_ref[...] + h1_ref[...]))


def _finalize(x, h0, h1):
    nspec = pl.BlockSpec((BN, D), lambda i: (i, 0))
    return pl.pallas_call(
        _final_body,
        grid=(N // BN,),
        in_specs=[nspec, nspec, nspec],
        out_specs=nspec,
        out_shape=jax.ShapeDtypeStruct((N, D), jnp.float32),
    )(x, h0, h1)


def kernel(x, edge_index, edge_feats,
           W_src_1, b_src_1, W_src_2, b_src_2,
           W_dst_1, b_dst_1, W_dst_2, b_dst_2,
           W_edge_1, b_edge_1, W_edge_2, b_edge_2,
           W_m_1, b_m_1, W_m_2, b_m_2):
    src = edge_index[0].reshape(NW, NCHUNK, CH)
    dst = edge_index[1].reshape(NW, NCHUNK, CH)
    b2 = lambda b: b.reshape(1, D)
    h_src, h_dst = _node_mlps(x, W_src_1, b2(b_src_1), W_src_2, b2(b_src_2),
                              W_dst_1, b2(b_dst_1), W_dst_2, b2(b_dst_2))
    g = _gather_mul_kernel()(h_src, h_dst, src, dst)
    m = _edge_mlps(edge_feats, g, W_edge_1, b2(b_edge_1), W_edge_2, b2(b_edge_2),
                   W_m_1, b2(b_m_1), W_m_2, b2(b_m_2))
    h_parts = _scatter_add_kernel()(m, dst)
    return _finalize(x, h_parts[:N], h_parts[NPAD:NPAD + N])


# speedup vs baseline: 1.1498x; 1.0020x over previous
import functools

import jax
import jax.numpy as jnp
from jax import lax
from jax.experimental import pallas as pl
from jax.experimental.pallas import tpu as pltpu
from jax.experimental.pallas import tpu_sc as plsc

N = 10000
E = 320000
D = 128
E_CHARGE = 1.602176634e-19
EPS0 = 8.8541878128e-12

NC = 2
NS = 16
NW = NC * NS
EPW = E // NW
CH = 80
NCHUNK = EPW // CH
NPAD = 10240
RPT = NPAD // NS
ZR = 64

_C_E2 = E_CHARGE ** 2
_C_4PIEPS = 4.0 * 3.141592653589793 * EPS0
_C_1EM10 = 1e-10


def _mish(v):
    u = jnp.exp(jnp.minimum(v, 30.0))
    t = u * (u + 2.0)
    return v * (t / (t + 2.0))


def _mlp(v, w1, b1, w2, b2):
    h = jnp.dot(v, w1, preferred_element_type=jnp.float32) + b1
    return jnp.dot(_mish(h), w2, preferred_element_type=jnp.float32) + b2


BN = 1000


def _node_body(x_ref, ws1, bs1, ws2, bs2, wd1, bd1, wd2, bd2, hs_ref, hd_ref):
    xb = x_ref[...]
    hs_ref[...] = _mlp(xb, ws1[...], bs1[...], ws2[...], bs2[...])
    hd_ref[...] = _mlp(xb, wd1[...], bd1[...], wd2[...], bd2[...])


def _node_mlps(x, ws1, bs1, ws2, bs2, wd1, bd1, wd2, bd2):
    wspec = pl.BlockSpec((D, D), lambda i: (0, 0))
    bspec = pl.BlockSpec((1, D), lambda i: (0, 0))
    nspec = pl.BlockSpec((BN, D), lambda i: (i, 0))
    return pl.pallas_call(
        _node_body,
        grid=(N // BN,),
        in_specs=[nspec, wspec, bspec, wspec, bspec, wspec, bspec, wspec, bspec],
        out_specs=[nspec, nspec],
        out_shape=[jax.ShapeDtypeStruct((N, D), jnp.float32)] * 2,
    )(x, ws1, bs1, ws2, bs2, wd1, bd1, wd2, bd2)


@functools.cache
def _sc_mesh():
    return plsc.VectorSubcoreMesh(
        core_axis_name="c", subcore_axis_name="s",
        num_cores=NC, num_subcores=NS)


@functools.cache
def _gather_mul_kernel():
    return functools.partial(
        pl.kernel,
        out_type=jax.ShapeDtypeStruct((E, D), jnp.float32),
        mesh=_sc_mesh(),
        scratch_types=[
            pltpu.VMEM((NCHUNK, CH), jnp.int32),
            pltpu.VMEM((NCHUNK, CH), jnp.int32),
            pltpu.VMEM((CH, D), jnp.float32),
            pltpu.VMEM((CH, D), jnp.float32),
            pltpu.VMEM((CH, D), jnp.float32),
            pltpu.VMEM((CH, D), jnp.float32),
            pltpu.VMEM((CH, D), jnp.float32),
            pltpu.VMEM((CH, D), jnp.float32),
            pltpu.SemaphoreType.DMA,
            pltpu.SemaphoreType.DMA,
            pltpu.SemaphoreType.DMA,
            pltpu.SemaphoreType.DMA,
        ],
    )(_gather_mul_body)


def _gather_mul_body(hsrc_hbm, hdst_hbm, src_hbm, dst_hbm, g_hbm,
                     ia_v, ib_v, ra0, rb0, ra1, rb1, wa0, wa1,
                     sem0, sem1, wsem0, wsem1):
    wid = lax.axis_index("s") * NC + lax.axis_index("c")
    base = wid * EPW
    pltpu.sync_copy(src_hbm.at[wid], ia_v)
    pltpu.sync_copy(dst_hbm.at[wid], ib_v)
    gbufs = ((ra0, rb0, sem0), (ra1, rb1, sem1))
    wbufs = ((wa0, wsem0), (wa1, wsem1))

    def fire(c, ra, rb, sem):
        pltpu.async_copy(hsrc_hbm.at[ia_v.at[c]], ra, sem)
        pltpu.async_copy(hdst_hbm.at[ib_v.at[c]], rb, sem)

    def process(c, p):
        ra, rb, sem = gbufs[p]
        wa, wsem = wbufs[p]
        pltpu.make_async_copy(hsrc_hbm.at[ia_v.at[c]], ra, sem).wait()
        pltpu.make_async_copy(hdst_hbm.at[ib_v.at[c]], rb, sem).wait()

        @pl.when(c >= 2)
        def _():
            pltpu.make_async_copy(
                wa, g_hbm.at[pl.ds(base + (c - 2) * CH, CH)], wsem).wait()

        @plsc.parallel_loop(0, CH, unroll=4)
        def _(i):
            for r in range(D // 16):
                sl = pl.ds(r * 16, 16)
                wa[i, sl] = ra[i, sl] * rb[i, sl]
        nc = c + 2

        @pl.when(nc < NCHUNK)
        def _():
            fire(nc, ra, rb, sem)

        pltpu.async_copy(wa, g_hbm.at[pl.ds(base + c * CH, CH)], wsem)

    fire(0, *gbufs[0])
    fire(1, *gbufs[1])

    def body(k, carry):
        for p in range(2):
            process(2 * k + p, p)
        return carry

    lax.fori_loop(0, NCHUNK // 2, body, 0)
    lastp = (NCHUNK - 1) % 2
    process(jnp.int32(NCHUNK - 1), lastp)
    for p in range(2):
        wa, wsem = wbufs[p]
        c = NCHUNK - 1 if p == lastp else NCHUNK - 2
        pltpu.make_async_copy(
            wa, g_hbm.at[pl.ds(base + c * CH, CH)], wsem).wait()


BE = 2000


def _edge_body(ef_ref, g_ref, we1, be1, we2, be2, wm1, bm1, wm2, bm2, m_ref):
    ef = _mlp(ef_ref[...], we1[...], be1[...], we2[...], be2[...])
    num = g_ref[...] * _C_E2
    den = (_C_4PIEPS * ef) * _C_1EM10
    hn = num / den
    m_ref[...] = _mlp(hn, wm1[...], bm1[...], wm2[...], bm2[...])


def _edge_mlps(edge_feats, g, we1, be1, we2, be2, wm1, bm1, wm2, bm2):
    wspec = pl.BlockSpec((D, D), lambda i: (0, 0))
    bspec = pl.BlockSpec((1, D), lambda i: (0, 0))
    espec = pl.BlockSpec((BE, D), lambda i: (i, 0))
    return pl.pallas_call(
        _edge_body,
        grid=(E // BE,),
        in_specs=[espec, espec, wspec, bspec, wspec, bspec,
                  wspec, bspec, wspec, bspec],
        out_specs=espec,
        out_shape=jax.ShapeDtypeStruct((E, D), jnp.float32),
    )(edge_feats, g, we1, be1, we2, be2, wm1, bm1, wm2, bm2)


@functools.cache
def _scatter_add_kernel():
    return functools.partial(
        pl.kernel,
        out_type=jax.ShapeDtypeStruct((NC * NPAD, D), jnp.float32),
        mesh=_sc_mesh(),
        scratch_types=[
            pltpu.VMEM((NCHUNK, CH), jnp.int32),
            pltpu.VMEM((CH, D), jnp.float32),
            pltpu.VMEM((CH, D), jnp.float32),
            pltpu.VMEM((ZR, D), jnp.float32),
            pltpu.VMEM_SHARED((NPAD, D), jnp.float32),
            pltpu.SemaphoreType.DMA,
            pltpu.SemaphoreType.DMA,
        ],
    )(_scatter_add_body)


def _scatter_add_body(m_hbm, dst_hbm, h_hbm, idx_v, rows0, rows1, zb_v,
                      shared, sem0, sem1):
    cid = lax.axis_index("c")
    sid = lax.axis_index("s")
    wid = sid * NC + cid
    base = wid * EPW
    pltpu.sync_copy(dst_hbm.at[wid], idx_v)
    bufs = ((rows0, sem0), (rows1, sem1))

    def fire(c, rows, sem):
        pltpu.async_copy(m_hbm.at[pl.ds(base + c * CH, CH)], rows, sem)

    def process(c, rows, sem):
        pltpu.make_async_copy(m_hbm.at[pl.ds(base + c * CH, CH)], rows,
                              sem).wait()
        pltpu.sync_copy(rows, shared.at[idx_v.at[c]], add=True)

    fire(0, *bufs[0])
    fire(1, *bufs[1])

    def zrow(i, carry):
        for r in range(D // 16):
            zb_v[i, pl.ds(r * 16, 16)] = jnp.zeros((16,), jnp.float32)
        return carry

    lax.fori_loop(0, ZR, zrow, 0)
    for k in range(RPT // ZR):
        pltpu.sync_copy(zb_v, shared.at[pl.ds(sid * RPT + k * ZR, ZR)])
    plsc.subcore_barrier()

    def chunk(k, carry):
        for p in range(2):
            c = 2 * k + p
            rows, sem = bufs[p]
            process(c, rows, sem)
            nc = c + 2

            @pl.when(nc < NCHUNK)
            def _():
                fire(nc, rows, sem)
        return carry

    lax.fori_loop(0, NCHUNK // 2, chunk, 0)
    process(NCHUNK - 1, *bufs[(NCHUNK - 1) % 2])
    plsc.subcore_barrier()
    pltpu.sync_copy(shared.at[pl.ds(sid * RPT, RPT)],
                    h_hbm.at[pl.ds(cid * NPAD + sid * RPT, RPT)])


def _final_body(x_ref, h0_ref, h1_ref, out_ref):
    out_ref[...] = jax.nn.softplus(x_ref[...] + (h0_ref[...] + h1_ref[...]))


def _finalize(x, h0, h1):
    nspec = pl.BlockSpec((BN, D), lambda i: (i, 0))
    return pl.pallas_call(
        _final_body,
        grid=(N // BN,),
        in_specs=[nspec, nspec, nspec],
        out_specs=nspec,
        out_shape=jax.ShapeDtypeStruct((N, D), jnp.float32),
    )(x, h0, h1)


def kernel(x, edge_index, edge_feats,
           W_src_1, b_src_1, W_src_2, b_src_2,
           W_dst_1, b_dst_1, W_dst_2, b_dst_2,
           W_edge_1, b_edge_1, W_edge_2, b_edge_2,
           W_m_1, b_m_1, W_m_2, b_m_2):
    src = edge_index[0].reshape(NW, NCHUNK, CH)
    dst = edge_index[1].reshape(NW, NCHUNK, CH)
    b2 = lambda b: b.reshape(1, D)
    h_src, h_dst = _node_mlps(x, W_src_1, b2(b_src_1), W_src_2, b2(b_src_2),
                              W_dst_1, b2(b_dst_1), W_dst_2, b2(b_dst_2))
    g = _gather_mul_kernel()(h_src, h_dst, src, dst)
    m = _edge_mlps(edge_feats, g, W_edge_1, b2(b_edge_1), W_edge_2, b2(b_edge_2),
                   W_m_1, b2(b_m_1), W_m_2, b2(b_m_2))
    h_parts = _scatter_add_kernel()(m, dst)
    return _finalize(x, h_parts[:N], h_parts[NPAD:NPAD + N])
